# baseline (speedup 1.0000x reference)
import jax
import jax.numpy as jnp
from jax.experimental import pallas as pl

N_BATCH = 32
N_NODE = 128
D_NODE = 512
D_EDGE = 16


def _mp_kernel(h_ref, a_ref, e_ref, ws_ref, wt_ref, wb_ref, b_ref, o_ref):
    h = h_ref[0]
    a = a_ref[0]
    e = e_ref[0]

    deg = jnp.sum(a, axis=1, keepdims=True)
    he_h = h * deg
    he_e = jnp.sum(a[:, :, None] * e, axis=1)

    acc = jnp.dot(h, ws_ref[...], preferred_element_type=jnp.float32)
    acc += jnp.dot(he_h, wt_ref[...], preferred_element_type=jnp.float32)
    acc += jnp.dot(he_e, wb_ref[...], preferred_element_type=jnp.float32)
    o_ref[0] = acc + b_ref[...]


def kernel(H, A, E, N, W_self, W_nei, bias):
    del N
    W_top = W_nei[:D_NODE]
    W_bot = W_nei[D_NODE:]
    bias2d = bias.reshape(1, D_NODE)

    out = pl.pallas_call(
        _mp_kernel,
        grid=(N_BATCH,),
        in_specs=[
            pl.BlockSpec((1, N_NODE, D_NODE), lambda b: (b, 0, 0)),
            pl.BlockSpec((1, N_NODE, N_NODE), lambda b: (b, 0, 0)),
            pl.BlockSpec((1, N_NODE, N_NODE, D_EDGE), lambda b: (b, 0, 0, 0)),
            pl.BlockSpec((D_NODE, D_NODE), lambda b: (0, 0)),
            pl.BlockSpec((D_NODE, D_NODE), lambda b: (0, 0)),
            pl.BlockSpec((D_EDGE, D_NODE), lambda b: (0, 0)),
            pl.BlockSpec((1, D_NODE), lambda b: (0, 0)),
        ],
        out_specs=pl.BlockSpec((1, N_NODE, D_NODE), lambda b: (b, 0, 0)),
        out_shape=jax.ShapeDtypeStruct((N_BATCH, N_NODE, D_NODE), jnp.float32),
    )(H, A, E, W_self, W_top, W_bot, bias2d)
    return out


# speedup vs baseline: 2.3809x; 2.3809x over previous
import jax
import jax.numpy as jnp
from jax import lax
from jax.experimental import pallas as pl

N_BATCH = 32
N_NODE = 128
D_NODE = 512
D_EDGE = 16
BB = 2
M = BB * N_NODE


def _mp_kernel(h_ref, a_ref, e_ref, wcat_ref, wtile_ref, r_ref, b_ref, o_ref):
    h = h_ref[...].reshape(M, D_NODE)
    a = a_ref[...].reshape(M, N_NODE)
    e2 = e_ref[...].reshape(M, N_NODE * D_EDGE)

    deg = jnp.sum(a, axis=1, keepdims=True)

    y = jnp.dot(h.astype(jnp.bfloat16), wcat_ref[...],
                preferred_element_type=jnp.float32)
    out = y[:, :D_NODE] + deg * y[:, D_NODE:]

    a_rep = jnp.dot(a.astype(jnp.bfloat16), r_ref[...],
                    preferred_element_type=jnp.float32)
    t = a_rep * e2
    t = t[:, :1024] + t[:, 1024:]
    t = t[:, :512] + t[:, 512:]
    t = t[:, :256] + t[:, 256:]
    t = t[:, :128] + t[:, 128:]
    out += jnp.dot(t.astype(jnp.bfloat16), wtile_ref[...],
                   preferred_element_type=jnp.float32)

    o_ref[...] = (out + b_ref[...]).reshape(BB, N_NODE, D_NODE)


def kernel(H, A, E, N, W_self, W_nei, bias):
    del N
    E2 = E.reshape(N_BATCH, N_NODE, N_NODE * D_EDGE)
    W_cat = jnp.concatenate([W_self, W_nei[:D_NODE]], axis=1).astype(jnp.bfloat16)
    W_tile = jnp.tile(W_nei[D_NODE:], (N_NODE // D_EDGE, 1)).astype(jnp.bfloat16)
    jj = lax.broadcasted_iota(jnp.int32, (N_NODE, N_NODE * D_EDGE), 0)
    mm = lax.broadcasted_iota(jnp.int32, (N_NODE, N_NODE * D_EDGE), 1)
    R = (mm // D_EDGE == jj).astype(jnp.bfloat16)
    bias2d = bias.reshape(1, D_NODE)

    grid = N_BATCH // BB
    out = pl.pallas_call(
        _mp_kernel,
        grid=(grid,),
        in_specs=[
            pl.BlockSpec((BB, N_NODE, D_NODE), lambda b: (b, 0, 0)),
            pl.BlockSpec((BB, N_NODE, N_NODE), lambda b: (b, 0, 0)),
            pl.BlockSpec((BB, N_NODE, N_NODE * D_EDGE), lambda b: (b, 0, 0)),
            pl.BlockSpec((D_NODE, 2 * D_NODE), lambda b: (0, 0)),
            pl.BlockSpec((N_NODE, D_NODE), lambda b: (0, 0)),
            pl.BlockSpec((N_NODE, N_NODE * D_EDGE), lambda b: (0, 0)),
            pl.BlockSpec((1, D_NODE), lambda b: (0, 0)),
        ],
        out_specs=pl.BlockSpec((BB, N_NODE, D_NODE), lambda b: (b, 0, 0)),
        out_shape=jax.ShapeDtypeStruct((N_BATCH, N_NODE, D_NODE), jnp.float32),
    )(H, A, E2, W_cat, W_tile, R, bias2d)
    return out


# speedup vs baseline: 3.2251x; 1.3546x over previous
import jax
import jax.numpy as jnp
from jax.experimental import pallas as pl

N_BATCH = 32
N_NODE = 128
D_NODE = 512
D_EDGE = 16
BB = 2
M = BB * N_NODE


def _dense_kernel(h_ref, a_ref, wcat_ref, b_ref, p_ref):
    h = h_ref[...].reshape(M, D_NODE)
    a = a_ref[...].reshape(M, N_NODE)
    deg = jnp.sum(a, axis=1, keepdims=True)
    y = jnp.dot(h.astype(jnp.bfloat16), wcat_ref[...],
                preferred_element_type=jnp.float32)
    p = y[:, :D_NODE] + deg * y[:, D_NODE:] + b_ref[...]
    p_ref[...] = p.reshape(BB, N_NODE, D_NODE)


def _edge_kernel(p_ref, a_ref, et_ref, wrep_ref, o_ref):
    slabs = []
    for bb in range(BB):
        a = a_ref[bb]
        slabs.append(jnp.concatenate(
            [et_ref[bb, c] * a for c in range(D_EDGE)], axis=1))
    t = jnp.concatenate(slabs, axis=0)
    out = jnp.dot(t.astype(jnp.bfloat16), wrep_ref[...],
                  preferred_element_type=jnp.float32)
    out += p_ref[...].reshape(M, D_NODE)
    o_ref[...] = out.reshape(BB, N_NODE, D_NODE)


def kernel(H, A, E, N, W_self, W_nei, bias):
    del N
    E_t = jnp.transpose(E, (0, 3, 1, 2))
    W_cat = jnp.concatenate([W_self, W_nei[:D_NODE]], axis=1).astype(jnp.bfloat16)
    W_rep = jnp.repeat(W_nei[D_NODE:], N_NODE, axis=0).astype(jnp.bfloat16)
    bias2d = bias.reshape(1, D_NODE)

    grid = N_BATCH // BB
    P = pl.pallas_call(
        _dense_kernel,
        grid=(grid,),
        in_specs=[
            pl.BlockSpec((BB, N_NODE, D_NODE), lambda b: (b, 0, 0)),
            pl.BlockSpec((BB, N_NODE, N_NODE), lambda b: (b, 0, 0)),
            pl.BlockSpec((D_NODE, 2 * D_NODE), lambda b: (0, 0)),
            pl.BlockSpec((1, D_NODE), lambda b: (0, 0)),
        ],
        out_specs=pl.BlockSpec((BB, N_NODE, D_NODE), lambda b: (b, 0, 0)),
        out_shape=jax.ShapeDtypeStruct((N_BATCH, N_NODE, D_NODE), jnp.float32),
    )(H, A, W_cat, bias2d)

    out = pl.pallas_call(
        _edge_kernel,
        grid=(grid,),
        in_specs=[
            pl.BlockSpec((BB, N_NODE, D_NODE), lambda b: (b, 0, 0)),
            pl.BlockSpec((BB, N_NODE, N_NODE), lambda b: (b, 0, 0)),
            pl.BlockSpec((BB, D_EDGE, N_NODE, N_NODE), lambda b: (b, 0, 0, 0)),
            pl.BlockSpec((D_EDGE * N_NODE, D_NODE), lambda b: (0, 0)),
        ],
        out_specs=pl.BlockSpec((BB, N_NODE, D_NODE), lambda b: (b, 0, 0)),
        out_shape=jax.ShapeDtypeStruct((N_BATCH, N_NODE, D_NODE), jnp.float32),
    )(P, A, E_t, W_rep)
    return out


# speedup vs baseline: 3.6992x; 1.1470x over previous
import jax
import jax.numpy as jnp
from jax.experimental import pallas as pl

N_BATCH = 32
N_NODE = 128
D_NODE = 512
D_EDGE = 16
BB = 2
M = BB * N_NODE


def _dense_kernel(h_ref, a_ref, wcat_ref, b_ref, p_ref):
    h = h_ref[...].reshape(M, D_NODE)
    a = a_ref[...].reshape(M, N_NODE)
    deg = jnp.sum(a, axis=1, keepdims=True)
    y = jnp.dot(h.astype(jnp.bfloat16), wcat_ref[...],
                preferred_element_type=jnp.float32)
    p = y[:, :D_NODE] + deg * y[:, D_NODE:] + b_ref[...]
    p_ref[...] = p.reshape(BB, N_NODE, D_NODE).astype(jnp.bfloat16)


def _edge_kernel(p_ref, a_ref, et_ref, wrep_ref, o_ref):
    slabs = []
    for bb in range(BB):
        a = a_ref[bb].astype(jnp.bfloat16)
        slabs.append(jnp.concatenate(
            [et_ref[bb, c] * a for c in range(D_EDGE)], axis=1))
    t = jnp.concatenate(slabs, axis=0)
    out = jnp.dot(t, wrep_ref[...],
                  preferred_element_type=jnp.float32)
    out += p_ref[...].reshape(M, D_NODE).astype(jnp.float32)
    o_ref[...] = out.reshape(BB, N_NODE, D_NODE)


def kernel(H, A, E, N, W_self, W_nei, bias):
    del N
    E_t = jnp.transpose(E, (0, 3, 1, 2)).astype(jnp.bfloat16)
    W_cat = jnp.concatenate([W_self, W_nei[:D_NODE]], axis=1).astype(jnp.bfloat16)
    W_rep = jnp.repeat(W_nei[D_NODE:], N_NODE, axis=0).astype(jnp.bfloat16)
    bias2d = bias.reshape(1, D_NODE)

    grid = N_BATCH // BB
    P = pl.pallas_call(
        _dense_kernel,
        grid=(grid,),
        in_specs=[
            pl.BlockSpec((BB, N_NODE, D_NODE), lambda b: (b, 0, 0)),
            pl.BlockSpec((BB, N_NODE, N_NODE), lambda b: (b, 0, 0)),
            pl.BlockSpec((D_NODE, 2 * D_NODE), lambda b: (0, 0)),
            pl.BlockSpec((1, D_NODE), lambda b: (0, 0)),
        ],
        out_specs=pl.BlockSpec((BB, N_NODE, D_NODE), lambda b: (b, 0, 0)),
        out_shape=jax.ShapeDtypeStruct((N_BATCH, N_NODE, D_NODE), jnp.bfloat16),
    )(H, A, W_cat, bias2d)

    out = pl.pallas_call(
        _edge_kernel,
        grid=(grid,),
        in_specs=[
            pl.BlockSpec((BB, N_NODE, D_NODE), lambda b: (b, 0, 0)),
            pl.BlockSpec((BB, N_NODE, N_NODE), lambda b: (b, 0, 0)),
            pl.BlockSpec((BB, D_EDGE, N_NODE, N_NODE), lambda b: (b, 0, 0, 0)),
            pl.BlockSpec((D_EDGE * N_NODE, D_NODE), lambda b: (0, 0)),
        ],
        out_specs=pl.BlockSpec((BB, N_NODE, D_NODE), lambda b: (b, 0, 0)),
        out_shape=jax.ShapeDtypeStruct((N_BATCH, N_NODE, D_NODE), jnp.float32),
    )(P, A, E_t, W_rep)
    return out
